# baseline (speedup 1.0000x reference)
import functools

import jax
import jax.numpy as jnp
from jax import lax
from jax.experimental import pallas as pl
from jax.experimental.pallas import tpu as pltpu
from jax.experimental.pallas import tpu_sc as plsc

NUM_CORES = 2
NUM_SUBCORES = 16
NUM_WORKERS = NUM_CORES * NUM_SUBCORES
CHUNK = 128


@functools.partial(jax.jit, static_argnames=("batch", "dim"))
def _lookup(user, item, user_weight, item_weight, *, batch, dim):
    b_per_w = batch // NUM_WORKERS
    n_chunks = b_per_w // CHUNK

    mesh = plsc.VectorSubcoreMesh(core_axis_name="c", subcore_axis_name="s")

    @functools.partial(
        pl.kernel,
        mesh=mesh,
        out_type=(
            jax.ShapeDtypeStruct((batch, dim), jnp.float32),
            jax.ShapeDtypeStruct((batch, dim), jnp.float32),
        ),
        scratch_types=[
            pltpu.VMEM((b_per_w,), jnp.int32),
            pltpu.VMEM((b_per_w,), jnp.int32),
            pltpu.VMEM((b_per_w, dim), jnp.float32),
            pltpu.VMEM((b_per_w, dim), jnp.float32),
            pltpu.SemaphoreType.DMA,
        ],
        compiler_params=pltpu.CompilerParams(use_tc_tiling_on_sc=False),
    )
    def gather_kernel(user_hbm, item_hbm, uw_hbm, iw_hbm,
                      out_u_hbm, out_i_hbm,
                      idx_u, idx_i, rows_u, rows_i, sem):
        wid = lax.axis_index("s") * NUM_CORES + lax.axis_index("c")
        base = wid * b_per_w
        pltpu.sync_copy(user_hbm.at[pl.ds(base, b_per_w)], idx_u)
        pltpu.sync_copy(item_hbm.at[pl.ds(base, b_per_w)], idx_i)
        copies = []
        for j in range(n_chunks):
            sl = pl.ds(j * CHUNK, CHUNK)
            copies.append(
                pltpu.async_copy(uw_hbm.at[idx_u.at[sl]], rows_u.at[sl], sem))
            copies.append(
                pltpu.async_copy(iw_hbm.at[idx_i.at[sl]], rows_i.at[sl], sem))
        for c in copies:
            c.wait()
        pltpu.sync_copy(rows_u, out_u_hbm.at[pl.ds(base, b_per_w)])
        pltpu.sync_copy(rows_i, out_i_hbm.at[pl.ds(base, b_per_w)])

    return gather_kernel(user, item, user_weight, item_weight)


def kernel(user, item, user_weight, item_weight):
    batch = user.shape[0]
    dim = user_weight.shape[1]
    user = user.astype(jnp.int32)
    item = item.astype(jnp.int32)
    return _lookup(user, item, user_weight, item_weight, batch=batch, dim=dim)
